# Initial kernel scaffold; baseline (speedup 1.0000x reference)
import jax
import jax.numpy as jnp
from jax.experimental import pallas as pl


def kernel(q, k, v):
    raise NotImplementedError("write your pallas kernel here")


# speedup vs baseline: 49.4306x; 49.4306x over previous
import functools

import jax
import jax.numpy as jnp
import numpy as np
from jax.experimental import pallas as pl
import jax.experimental.pallas.tpu as pltpu

FRAG = 64
HALF = FRAG // 2
G = 6
P = 24
PPAD = 32
GPAD = 8
KQ = 5
TOP_U = 16
W_MEAN, W_MAX, W_TOPK, W_STD = 1.0, 0.6, 0.4, 0.2
ALPHA = 0.15
TAU = 8.0
NEG = -1e30


def _chooser_kernel(q_ref, k_ref, out_ref, *, T, d):
    kh = k_ref[0]
    qh = q_ref[0]
    knorm = jnp.sqrt(jnp.sum(kh * kh, axis=1, keepdims=True))
    kbar = kh / jnp.maximum(knorm, 1e-6)

    r32 = jax.lax.broadcasted_iota(jnp.int32, (PPAD, 1), 0)
    step = float(T - 1) / float(P - 1)
    idxp = jnp.where(r32 < P,
                     jnp.round(r32.astype(jnp.float32) * step).astype(jnp.int32),
                     -1)
    colT = jax.lax.broadcasted_iota(jnp.int32, (PPAD, T), 1)
    onehot_p = (colT == idxp).astype(jnp.float32)
    qp = jax.lax.dot_general(onehot_p, qh, (((1,), (0,)), ((), ())),
                             preferred_element_type=jnp.float32)
    qnorm = jnp.sqrt(jnp.sum(qp * qp, axis=1, keepdims=True))
    qp = qp / jnp.maximum(qnorm, 1e-6)

    s = jax.lax.dot_general(kbar, qp, (((1,), (1,)), ((), ())),
                            preferred_element_type=jnp.float32)
    s = jnp.maximum(s, 0.0)

    mean = jnp.sum(s, axis=1, keepdims=True) / P
    mx = jnp.max(s, axis=1, keepdims=True)
    ssq = jnp.sum(s * s, axis=1, keepdims=True)
    std = jnp.sqrt(jnp.maximum(ssq / P - mean * mean, 0.0))

    lane = jax.lax.broadcasted_iota(jnp.int32, (T, PPAD), 1)
    cur = s
    acc = jnp.zeros((T, 1), jnp.float32)
    for _ in range(KQ):
        m5 = jnp.max(cur, axis=1, keepdims=True)
        acc = acc + m5
        first = jnp.min(jnp.where(cur >= m5, lane, PPAD), axis=1, keepdims=True)
        cur = jnp.where(lane == first, -1.0, cur)
    topk_mean = acc / KQ

    u = W_MEAN * mean + W_MAX * mx + W_TOPK * topk_mean + W_STD * std

    row = jax.lax.broadcasted_iota(jnp.int32, (T, 1), 0)
    ucur = u
    masks = []
    idxvals = []
    for _ in range(TOP_U):
        mu = jnp.max(ucur)
        fid = jnp.min(jnp.where(ucur >= mu, row, T))
        oh = row == fid
        masks.append(oh.astype(jnp.float32))
        idxvals.append(fid.astype(jnp.float32).reshape(1, 1))
        ucur = jnp.where(oh, NEG, ucur)
    topmask = jnp.concatenate(masks, axis=1)
    idxv = jnp.concatenate(idxvals, axis=0)

    s_sub = jax.lax.dot_general(topmask, s, (((0,), (0,)), ((), ())),
                                preferred_element_type=jnp.float32)

    m_row = jnp.zeros((1, PPAD), jnp.float32)
    blocked = jnp.zeros((TOP_U, 1), jnp.bool_)
    r16 = jax.lax.broadcasted_iota(jnp.int32, (TOP_U, 1), 0)
    chosen_vals = []
    for _ in range(G):
        gains = jnp.sum(jnp.maximum(s_sub - m_row, 0.0), axis=1, keepdims=True)
        gains = jnp.where(blocked, -1e9, gains)
        mg = jnp.max(gains)
        j = jnp.min(jnp.where(gains >= mg, r16, TOP_U))
        ohj = r16 == j
        blocked = blocked | ohj
        chosen_vals.append(jnp.sum(jnp.where(ohj, idxv, 0.0)).reshape(1, 1))
        sel = jnp.sum(jnp.where(ohj, s_sub, 0.0), axis=0, keepdims=True)
        m_row = jnp.maximum(m_row, sel)
    chosen_row = jnp.concatenate(
        chosen_vals + [jnp.zeros((1, GPAD - G), jnp.float32)], axis=1)
    out_ref[0] = chosen_row.astype(jnp.int32)


def _attn_kernel(chosen_ref, q_ref, k_ref, v_ref, o_ref, *, T, d, C, HALO):
    h = pl.program_id(0)
    b = pl.program_id(1)
    s0 = b * C
    halo_start = jnp.clip(s0 - HALF, 0, T - HALO)

    qb = q_ref[0]
    k_halo = k_ref[0, pl.ds(halo_start, HALO), :]
    v_halo = v_ref[0, pl.ds(halo_start, HALO), :]

    inv_sqrt_d = 1.0 / float(np.sqrt(d))
    sl = jax.lax.dot_general(qb, k_halo, (((1,), (1,)), ((), ())),
                             preferred_element_type=jnp.float32) * inv_sqrt_d
    t_abs = s0 + jax.lax.broadcasted_iota(jnp.int32, (C, HALO), 0)
    j_abs = halo_start + jax.lax.broadcasted_iota(jnp.int32, (C, HALO), 1)
    start_t = jnp.clip(t_abs - HALF, 0, T - FRAG)
    valid = (j_abs >= start_t) & (j_abs < start_t + FRAG)
    prior = jnp.exp(jnp.abs(j_abs - t_abs).astype(jnp.float32) * (-1.0 / TAU))
    sl = jnp.where(valid, sl + ALPHA * prior, NEG)

    krows = []
    vrows = []
    cvals = []
    for i in range(G):
        ci = chosen_ref[h, i]
        krows.append(k_ref[0, pl.ds(ci, 1), :])
        vrows.append(v_ref[0, pl.ds(ci, 1), :])
        cvals.append(ci.reshape(1, 1))
    zpad = jnp.zeros((GPAD - G, d), jnp.float32)
    kg = jnp.concatenate(krows + [zpad], axis=0)
    vg = jnp.concatenate(vrows + [zpad], axis=0)
    crow = jnp.concatenate(
        cvals + [jnp.zeros((1, GPAD - G), jnp.int32)], axis=1)

    sg = jax.lax.dot_general(qb, kg, (((1,), (1,)), ((), ())),
                             preferred_element_type=jnp.float32) * inv_sqrt_d
    tq = s0 + jax.lax.broadcasted_iota(jnp.int32, (C, GPAD), 0)
    lane8 = jax.lax.broadcasted_iota(jnp.int32, (C, GPAD), 1)
    priorg = jnp.exp(jnp.abs(crow - tq).astype(jnp.float32) * (-1.0 / TAU))
    sg = jnp.where(lane8 < G, sg + ALPHA * priorg, NEG)

    mm = jnp.maximum(jnp.max(sl, axis=1, keepdims=True),
                     jnp.max(sg, axis=1, keepdims=True))
    pl_ = jnp.exp(sl - mm)
    pg = jnp.exp(sg - mm)
    denom = jnp.sum(pl_, axis=1, keepdims=True) + jnp.sum(pg, axis=1, keepdims=True)
    ctx = (jnp.dot(pl_, v_halo, preferred_element_type=jnp.float32)
           + jnp.dot(pg, vg, preferred_element_type=jnp.float32)) / denom
    o_ref[0] = ctx


@jax.jit
def kernel(q, k, v):
    B, H, T, d = q.shape
    BH = B * H
    qf = q.reshape(BH, T, d)
    kf = k.reshape(BH, T, d)
    vf = v.reshape(BH, T, d)

    chosen3 = pl.pallas_call(
        functools.partial(_chooser_kernel, T=T, d=d),
        grid=(BH,),
        in_specs=[
            pl.BlockSpec((1, T, d), lambda h: (h, 0, 0)),
            pl.BlockSpec((1, T, d), lambda h: (h, 0, 0)),
        ],
        out_specs=pl.BlockSpec((1, 1, GPAD), lambda h: (h, 0, 0)),
        out_shape=jax.ShapeDtypeStruct((BH, 1, GPAD), jnp.int32),
        compiler_params=pltpu.CompilerParams(
            dimension_semantics=("arbitrary",)),
    )(qf, kf)
    chosen = chosen3.reshape(BH, GPAD)

    C = 256
    HALO = C + FRAG
    NB = T // C
    grid_spec = pltpu.PrefetchScalarGridSpec(
        num_scalar_prefetch=1,
        grid=(BH, NB),
        in_specs=[
            pl.BlockSpec((1, C, d), lambda h, b, ch: (h, b, 0)),
            pl.BlockSpec((1, T, d), lambda h, b, ch: (h, 0, 0)),
            pl.BlockSpec((1, T, d), lambda h, b, ch: (h, 0, 0)),
        ],
        out_specs=pl.BlockSpec((1, C, d), lambda h, b, ch: (h, b, 0)),
    )
    ctx = pl.pallas_call(
        functools.partial(_attn_kernel, T=T, d=d, C=C, HALO=HALO),
        grid_spec=grid_spec,
        out_shape=jax.ShapeDtypeStruct((BH, T, d), jnp.float32),
        compiler_params=pltpu.CompilerParams(
            dimension_semantics=("arbitrary", "arbitrary")),
    )(chosen, qf, kf, vf)
    return ctx.reshape(B, H, T, d)


# speedup vs baseline: 67.7721x; 1.3711x over previous
import functools

import jax
import jax.numpy as jnp
import numpy as np
from jax.experimental import pallas as pl
import jax.experimental.pallas.tpu as pltpu

FRAG = 64
HALF = FRAG // 2
G = 6
P = 24
PPAD = 32
GPAD = 8
KQ = 5
TOP_U = 16
W_MEAN, W_MAX, W_TOPK, W_STD = 1.0, 0.6, 0.4, 0.2
ALPHA = 0.15
TAU = 8.0
NEG = -1e30


def _chooser_kernel(q_ref, k_ref, out_ref, *, T, d):
    kh = k_ref[0]
    qh = q_ref[0]

    r32 = jax.lax.broadcasted_iota(jnp.int32, (PPAD, 1), 0)
    step = float(T - 1) / float(P - 1)
    idxp = jnp.where(r32 < P,
                     jnp.round(r32.astype(jnp.float32) * step).astype(jnp.int32),
                     -1)
    colT = jax.lax.broadcasted_iota(jnp.int32, (PPAD, T), 1)
    onehot_p = (colT == idxp).astype(jnp.float32)
    qp = jax.lax.dot_general(onehot_p, qh, (((1,), (0,)), ((), ())),
                             preferred_element_type=jnp.float32)
    qnorm = jnp.sqrt(jnp.sum(qp * qp, axis=1, keepdims=True))
    qp = qp / jnp.maximum(qnorm, 1e-6)

    ones_row = jnp.full((1, d), 1.0, dtype=jnp.float32)
    knorm2 = jax.lax.dot_general(ones_row, kh * kh, (((1,), (1,)), ((), ())),
                                 preferred_element_type=jnp.float32)
    kinv = 1.0 / jnp.maximum(jnp.sqrt(knorm2), 1e-6)

    st = jax.lax.dot_general(qp, kh, (((1,), (1,)), ((), ())),
                             preferred_element_type=jnp.float32)
    st = jnp.maximum(st, 0.0) * kinv

    mean = jnp.sum(st, axis=0, keepdims=True) / P
    mx = jnp.max(st, axis=0, keepdims=True)
    ssq = jnp.sum(st * st, axis=0, keepdims=True)
    std = jnp.sqrt(jnp.maximum(ssq / P - mean * mean, 0.0))

    sub = jax.lax.broadcasted_iota(jnp.int32, (PPAD, T), 0)
    cur = st
    acc = jnp.zeros((1, T), jnp.float32)
    for _ in range(KQ):
        m5 = jnp.max(cur, axis=0, keepdims=True)
        acc = acc + m5
        first = jnp.min(jnp.where(cur >= m5, sub, PPAD), axis=0, keepdims=True)
        cur = jnp.where(sub == first, -1.0, cur)
    topk_mean = acc / KQ

    u = W_MEAN * mean + W_MAX * mx + W_TOPK * topk_mean + W_STD * std

    lane = jax.lax.broadcasted_iota(jnp.int32, (1, T), 1)
    ucur = u
    masks = []
    idxvals = []
    for _ in range(TOP_U):
        mu = jnp.max(ucur)
        fid = jnp.min(jnp.where(ucur >= mu, lane, T))
        oh = lane == fid
        masks.append(oh.astype(jnp.float32))
        idxvals.append(fid.astype(jnp.float32).reshape(1, 1))
        ucur = jnp.where(oh, NEG, ucur)
    topmask = jnp.concatenate(masks, axis=0)
    idxv = jnp.concatenate(idxvals, axis=1)

    s_sub = jax.lax.dot_general(st, topmask, (((1,), (1,)), ((), ())),
                                preferred_element_type=jnp.float32)

    m_col = jnp.zeros((PPAD, 1), jnp.float32)
    blocked = jnp.zeros((1, TOP_U), jnp.bool_)
    l16 = jax.lax.broadcasted_iota(jnp.int32, (1, TOP_U), 1)
    chosen_vals = []
    for _ in range(G):
        gains = jnp.sum(jnp.maximum(s_sub - m_col, 0.0), axis=0, keepdims=True)
        gains = jnp.where(blocked, -1e9, gains)
        mg = jnp.max(gains)
        j = jnp.min(jnp.where(gains >= mg, l16, TOP_U))
        ohj = l16 == j
        blocked = blocked | ohj
        chosen_vals.append(jnp.sum(jnp.where(ohj, idxv, 0.0)).reshape(1, 1))
        sel = jnp.sum(jnp.where(ohj, s_sub, 0.0), axis=1, keepdims=True)
        m_col = jnp.maximum(m_col, sel)
    chosen_row = jnp.concatenate(
        chosen_vals + [jnp.zeros((1, GPAD - G), jnp.float32)], axis=1)
    out_ref[0] = chosen_row.astype(jnp.int32)


def _attn_kernel(chosen_ref, q_ref, k_ref, v_ref, o_ref, *, T, d, C, HALO):
    h = pl.program_id(0)
    b = pl.program_id(1)
    s0 = b * C
    halo_start = jnp.clip(s0 - HALF, 0, T - HALO)

    qb = q_ref[0]
    k_halo = k_ref[0, pl.ds(halo_start, HALO), :]
    v_halo = v_ref[0, pl.ds(halo_start, HALO), :]

    inv_sqrt_d = 1.0 / float(np.sqrt(d))
    sl = jax.lax.dot_general(qb, k_halo, (((1,), (1,)), ((), ())),
                             preferred_element_type=jnp.float32) * inv_sqrt_d
    t_abs = s0 + jax.lax.broadcasted_iota(jnp.int32, (C, HALO), 0)
    j_abs = halo_start + jax.lax.broadcasted_iota(jnp.int32, (C, HALO), 1)
    start_t = jnp.clip(t_abs - HALF, 0, T - FRAG)
    valid = (j_abs >= start_t) & (j_abs < start_t + FRAG)
    prior = jnp.exp(jnp.abs(j_abs - t_abs).astype(jnp.float32) * (-1.0 / TAU))
    sl = jnp.where(valid, sl + ALPHA * prior, NEG)

    krows = []
    vrows = []
    cvals = []
    for i in range(G):
        ci = chosen_ref[h, i]
        krows.append(k_ref[0, pl.ds(ci, 1), :])
        vrows.append(v_ref[0, pl.ds(ci, 1), :])
        cvals.append(ci.reshape(1, 1))
    zpad = jnp.zeros((GPAD - G, d), jnp.float32)
    kg = jnp.concatenate(krows + [zpad], axis=0)
    vg = jnp.concatenate(vrows + [zpad], axis=0)
    crow = jnp.concatenate(
        cvals + [jnp.zeros((1, GPAD - G), jnp.int32)], axis=1)

    sg = jax.lax.dot_general(qb, kg, (((1,), (1,)), ((), ())),
                             preferred_element_type=jnp.float32) * inv_sqrt_d
    tq = s0 + jax.lax.broadcasted_iota(jnp.int32, (C, GPAD), 0)
    lane8 = jax.lax.broadcasted_iota(jnp.int32, (C, GPAD), 1)
    priorg = jnp.exp(jnp.abs(crow - tq).astype(jnp.float32) * (-1.0 / TAU))
    sg = jnp.where(lane8 < G, sg + ALPHA * priorg, NEG)

    mm = jnp.maximum(jnp.max(sl, axis=1, keepdims=True),
                     jnp.max(sg, axis=1, keepdims=True))
    pl_ = jnp.exp(sl - mm)
    pg = jnp.exp(sg - mm)
    denom = jnp.sum(pl_, axis=1, keepdims=True) + jnp.sum(pg, axis=1, keepdims=True)
    ctx = (jnp.dot(pl_, v_halo, preferred_element_type=jnp.float32)
           + jnp.dot(pg, vg, preferred_element_type=jnp.float32)) / denom
    o_ref[0] = ctx


@jax.jit
def kernel(q, k, v):
    B, H, T, d = q.shape
    BH = B * H
    qf = q.reshape(BH, T, d)
    kf = k.reshape(BH, T, d)
    vf = v.reshape(BH, T, d)

    chosen3 = pl.pallas_call(
        functools.partial(_chooser_kernel, T=T, d=d),
        grid=(BH,),
        in_specs=[
            pl.BlockSpec((1, T, d), lambda h: (h, 0, 0)),
            pl.BlockSpec((1, T, d), lambda h: (h, 0, 0)),
        ],
        out_specs=pl.BlockSpec((1, 1, GPAD), lambda h: (h, 0, 0)),
        out_shape=jax.ShapeDtypeStruct((BH, 1, GPAD), jnp.int32),
        compiler_params=pltpu.CompilerParams(
            dimension_semantics=("arbitrary",)),
    )(qf, kf)
    chosen = chosen3.reshape(BH, GPAD)

    C = 256
    HALO = C + FRAG
    NB = T // C
    grid_spec = pltpu.PrefetchScalarGridSpec(
        num_scalar_prefetch=1,
        grid=(BH, NB),
        in_specs=[
            pl.BlockSpec((1, C, d), lambda h, b, ch: (h, b, 0)),
            pl.BlockSpec((1, T, d), lambda h, b, ch: (h, 0, 0)),
            pl.BlockSpec((1, T, d), lambda h, b, ch: (h, 0, 0)),
        ],
        out_specs=pl.BlockSpec((1, C, d), lambda h, b, ch: (h, b, 0)),
    )
    ctx = pl.pallas_call(
        functools.partial(_attn_kernel, T=T, d=d, C=C, HALO=HALO),
        grid_spec=grid_spec,
        out_shape=jax.ShapeDtypeStruct((BH, T, d), jnp.float32),
        compiler_params=pltpu.CompilerParams(
            dimension_semantics=("arbitrary", "arbitrary")),
    )(chosen, qf, kf, vf)
    return ctx.reshape(B, H, T, d)
